# baseline (speedup 1.0000x reference)
import functools

import jax
import jax.numpy as jnp
from jax import lax
from jax.experimental import pallas as pl
from jax.experimental.pallas import tpu as pltpu
from jax.experimental.pallas import tpu_sc as plsc

CARDS_NUM = 1000000
HALF = CARDS_NUM // 2
EMB_DIM = 64
HIDDEN = 128
STATE = 64
BATCH = 4096
HIST = 200

N = BATCH * HIST
NW = 32
B_PER_W = N // NW
CH = 1024
N_CH = B_PER_W // CH
LANES = 16

ROWS_BLK = 5000
N_BLK = HALF // ROWS_BLK


def _mlp2(x, w1, b1, w2, b2):
    h = jax.nn.sigmoid(jnp.dot(x, w1, preferred_element_type=jnp.float32) + b1)
    return jax.nn.sigmoid(jnp.dot(h, w2, preferred_element_type=jnp.float32) + b2)


def _table_mlp_body(table_hbm, w1_ref, b1_ref, w2_ref, b2_ref, o_ref,
                    lo_buf, hi_buf, sems):
    i = pl.program_id(0)

    def start(j, slot):
        pltpu.make_async_copy(
            table_hbm.at[pl.ds(j * ROWS_BLK, ROWS_BLK)],
            lo_buf.at[slot], sems.at[slot, 0]).start()
        pltpu.make_async_copy(
            table_hbm.at[pl.ds(HALF + j * ROWS_BLK, ROWS_BLK)],
            hi_buf.at[slot], sems.at[slot, 1]).start()

    @pl.when(i == 0)
    def _():
        start(0, 0)

    @pl.when(i + 1 < N_BLK)
    def _():
        start(i + 1, (i + 1) % 2)

    slot = i % 2
    pltpu.make_async_copy(
        table_hbm.at[pl.ds(i * ROWS_BLK, ROWS_BLK)],
        lo_buf.at[slot], sems.at[slot, 0]).wait()
    pltpu.make_async_copy(
        table_hbm.at[pl.ds(HALF + i * ROWS_BLK, ROWS_BLK)],
        hi_buf.at[slot], sems.at[slot, 1]).wait()

    w1, b1, w2, b2 = w1_ref[...], b1_ref[...], w2_ref[...], b2_ref[...]
    o_ref[:, 0:STATE] = _mlp2(lo_buf[slot], w1, b1, w2, b2)
    o_ref[:, STATE:2 * STATE] = _mlp2(hi_buf[slot], w1, b1, w2, b2)


def _table_mlp(table, W1, b1, W2, b2):
    return pl.pallas_call(
        _table_mlp_body,
        grid=(N_BLK,),
        in_specs=[
            pl.BlockSpec(memory_space=pl.ANY),
            pl.BlockSpec((EMB_DIM, HIDDEN), lambda i: (0, 0)),
            pl.BlockSpec((1, HIDDEN), lambda i: (0, 0)),
            pl.BlockSpec((HIDDEN, STATE), lambda i: (0, 0)),
            pl.BlockSpec((1, STATE), lambda i: (0, 0)),
        ],
        out_specs=pl.BlockSpec((ROWS_BLK, 2 * STATE), lambda i: (i, 0)),
        out_shape=jax.ShapeDtypeStruct((HALF, 2 * STATE), jnp.float32),
        scratch_shapes=[
            pltpu.VMEM((2, ROWS_BLK, EMB_DIM), jnp.float32),
            pltpu.VMEM((2, ROWS_BLK, EMB_DIM), jnp.float32),
            pltpu.SemaphoreType.DMA((2, 2)),
        ],
    )(table, W1, b1, W2, b2)


def _gather_body(table_hbm, idx_hbm, out_hbm, idx_v, rows_v, sem):
    core = lax.axis_index("c")
    sub = lax.axis_index("s")
    wid = sub * 2 + core
    base = wid * B_PER_W
    pltpu.sync_copy(idx_hbm.at[pl.ds(base, B_PER_W)], idx_v)

    def remap(j, _):
        v = idx_v[pl.ds(j * LANES, LANES)]
        ge = v >= HALF
        v2 = jnp.where(ge, 2 * (v - HALF) + 1, 2 * v)
        idx_v[pl.ds(j * LANES, LANES)] = v2
        return ()

    lax.fori_loop(0, B_PER_W // LANES, remap, (), unroll=8)

    def chunk(c, _):
        off = c * CH
        pltpu.async_copy(
            table_hbm.at[idx_v.at[pl.ds(off, CH)]], rows_v, sem
        ).wait()
        pltpu.sync_copy(rows_v, out_hbm.at[pl.ds(base + off, CH)])
        return ()

    lax.fori_loop(0, N_CH, chunk, (), unroll=False)


def _sc_gather(table, idx):
    mesh = plsc.VectorSubcoreMesh(core_axis_name="c", subcore_axis_name="s")
    return pl.kernel(
        _gather_body,
        out_type=jax.ShapeDtypeStruct((N, STATE), jnp.float32),
        mesh=mesh,
        scratch_types=[
            pltpu.VMEM((B_PER_W,), jnp.int32),
            pltpu.VMEM((CH, STATE), jnp.float32),
            pltpu.SemaphoreType.DMA,
        ],
        compiler_params=pltpu.CompilerParams(use_tc_tiling_on_sc=False),
    )(table, idx)


@jax.jit
def kernel(cards_id, card_embedding, W1, b1, W2, b2):
    idx = cards_id.reshape(-1).astype(jnp.int32)
    packed = _table_mlp(
        card_embedding, W1, b1.reshape(1, HIDDEN), W2, b2.reshape(1, STATE)
    )
    tview = packed.reshape(CARDS_NUM, STATE)
    out = _sc_gather(tview, idx)
    out = jax.lax.optimization_barrier(out.reshape(N // 2, 2 * STATE))
    return out.reshape(BATCH, HIST, STATE)


# speedup vs baseline: 1.2198x; 1.2198x over previous
import functools

import jax
import jax.numpy as jnp
from jax import lax
from jax.experimental import pallas as pl
from jax.experimental.pallas import tpu as pltpu
from jax.experimental.pallas import tpu_sc as plsc

CARDS_NUM = 1000000
HALF = CARDS_NUM // 2
EMB_DIM = 64
HIDDEN = 128
STATE = 64
BATCH = 4096
HIST = 200

N = BATCH * HIST
NW = 32
B_PER_W = N // NW
CH = 1024
N_CH = B_PER_W // CH

CBLK = 4096
N_BLK = (CARDS_NUM + 2 * CBLK - 1) // (2 * CBLK)
P_ROWS = N_BLK * CBLK


def _mlp2_t(x_t, w1, b1c, w2, b2c, eye):
    dn = (((0,), (0,)), ((), ()))
    h_t = jax.nn.sigmoid(
        lax.dot_general(w1, x_t, dn, preferred_element_type=jnp.float32)
        + b1c
    )
    s_t = jax.nn.sigmoid(
        lax.dot_general(w2, h_t, dn, preferred_element_type=jnp.float32)
        + b2c
    )
    return lax.dot_general(s_t, eye, dn, preferred_element_type=jnp.float32)


def _table_mlp_body(lo_ref, hi_ref, w1_ref, b1_ref, w2_ref, b2_ref, o_ref):
    w1, w2 = w1_ref[...], w2_ref[...]
    b1c, b2c = b1_ref[...], b2_ref[...]
    eye = jnp.eye(STATE, dtype=jnp.float32)
    o_ref[:, 0:STATE] = _mlp2_t(lo_ref[...], w1, b1c, w2, b2c, eye)
    o_ref[:, STATE:2 * STATE] = _mlp2_t(hi_ref[...], w1, b1c, w2, b2c, eye)


def _table_mlp(xt, W1, b1c, W2, b2c):
    return pl.pallas_call(
        _table_mlp_body,
        grid=(N_BLK,),
        in_specs=[
            pl.BlockSpec((EMB_DIM, CBLK), lambda i: (0, 2 * i)),
            pl.BlockSpec(
                (EMB_DIM, CBLK),
                lambda i: (0, jnp.minimum(2 * i + 1, 2 * N_BLK - 2)),
            ),
            pl.BlockSpec((EMB_DIM, HIDDEN), lambda i: (0, 0)),
            pl.BlockSpec((HIDDEN, 1), lambda i: (0, 0)),
            pl.BlockSpec((HIDDEN, STATE), lambda i: (0, 0)),
            pl.BlockSpec((STATE, 1), lambda i: (0, 0)),
        ],
        out_specs=pl.BlockSpec((CBLK, 2 * STATE), lambda i: (i, 0)),
        out_shape=jax.ShapeDtypeStruct((P_ROWS, 2 * STATE), jnp.float32),
    )(xt, xt, W1, b1c, W2, b2c)


REMAP_BLK = 512


def _remap_body(i_ref, o_ref):
    t = i_ref[...]
    pair = t >> 13
    o = t & (2 * CBLK - 1)
    h = o >> 12
    j = o & (CBLK - 1)
    o_ref[...] = (pair << 13) + 2 * j + h


def _remap(cards_id):
    return pl.pallas_call(
        _remap_body,
        grid=(BATCH // REMAP_BLK,),
        in_specs=[pl.BlockSpec((REMAP_BLK, HIST), lambda i: (i, 0))],
        out_specs=pl.BlockSpec((REMAP_BLK, HIST), lambda i: (i, 0)),
        out_shape=jax.ShapeDtypeStruct((BATCH, HIST), jnp.int32),
    )(cards_id)


def _gather_body(table_hbm, idx_hbm, out_hbm, idx_v, rows_v, sem):
    core = lax.axis_index("c")
    sub = lax.axis_index("s")
    wid = sub * 2 + core
    base = wid * B_PER_W
    pltpu.sync_copy(idx_hbm.at[pl.ds(base, B_PER_W)], idx_v)

    def chunk(c, _):
        off = c * CH
        pltpu.async_copy(
            table_hbm.at[idx_v.at[pl.ds(off, CH)]], rows_v, sem
        ).wait()
        pltpu.sync_copy(rows_v, out_hbm.at[pl.ds(base + off, CH)])
        return ()

    lax.fori_loop(0, N_CH, chunk, (), unroll=False)


def _sc_gather(table, idx):
    mesh = plsc.VectorSubcoreMesh(core_axis_name="c", subcore_axis_name="s")
    return pl.kernel(
        _gather_body,
        out_type=jax.ShapeDtypeStruct((N, STATE), jnp.float32),
        mesh=mesh,
        scratch_types=[
            pltpu.VMEM((B_PER_W,), jnp.int32),
            pltpu.VMEM((CH, STATE), jnp.float32),
            pltpu.SemaphoreType.DMA,
        ],
        compiler_params=pltpu.CompilerParams(use_tc_tiling_on_sc=False),
    )(table, idx)


@jax.jit
def kernel(cards_id, card_embedding, W1, b1, W2, b2):
    xt = jnp.swapaxes(card_embedding, 0, 1)
    packed = _table_mlp(
        xt, W1, b1.reshape(HIDDEN, 1), W2, b2.reshape(STATE, 1)
    )
    tview = packed.reshape(2 * P_ROWS, STATE)
    idx = _remap(cards_id.astype(jnp.int32)).reshape(-1)
    out = _sc_gather(tview, idx)
    out = jax.lax.optimization_barrier(out.reshape(N // 2, 2 * STATE))
    return out.reshape(BATCH, HIST, STATE)


# speedup vs baseline: 1.2279x; 1.0067x over previous
import functools

import jax
import jax.numpy as jnp
from jax import lax
from jax.experimental import pallas as pl
from jax.experimental.pallas import tpu as pltpu
from jax.experimental.pallas import tpu_sc as plsc
from jax.experimental import layout as jax_layout

CARDS_NUM = 1000000
HALF = CARDS_NUM // 2
EMB_DIM = 64
HIDDEN = 128
STATE = 64
BATCH = 4096
HIST = 200

N = BATCH * HIST
NW = 32
B_PER_W = N // NW
CH = 1024
N_CH = B_PER_W // CH

CBLK = 8192
CSH = 13
N_BLK = (CARDS_NUM + 2 * CBLK - 1) // (2 * CBLK)
P_ROWS = N_BLK * CBLK


def _mlp2_t(x_t, w1, b1c, w2, b2c, eye):
    dn = (((0,), (0,)), ((), ()))
    h_t = jax.nn.sigmoid(
        lax.dot_general(w1, x_t, dn, preferred_element_type=jnp.float32)
        + b1c
    )
    s_t = jax.nn.sigmoid(
        lax.dot_general(w2, h_t, dn, preferred_element_type=jnp.float32)
        + b2c
    )
    return lax.dot_general(s_t, eye, dn, preferred_element_type=jnp.float32)


def _table_mlp_body(lo_ref, hi_ref, w1_ref, b1_ref, w2_ref, b2_ref, o_ref):
    w1, w2 = w1_ref[...], w2_ref[...]
    b1c, b2c = b1_ref[...], b2_ref[...]
    eye = jnp.eye(STATE, dtype=jnp.float32)
    o_ref[:, 0:STATE] = _mlp2_t(lo_ref[...], w1, b1c, w2, b2c, eye)
    o_ref[:, STATE:2 * STATE] = _mlp2_t(hi_ref[...], w1, b1c, w2, b2c, eye)


def _table_mlp(xt, W1, b1c, W2, b2c):
    return pl.pallas_call(
        _table_mlp_body,
        grid=(N_BLK,),
        in_specs=[
            pl.BlockSpec((EMB_DIM, CBLK), lambda i: (0, 2 * i)),
            pl.BlockSpec(
                (EMB_DIM, CBLK),
                lambda i: (0, jnp.minimum(2 * i + 1, 2 * N_BLK - 2)),
            ),
            pl.BlockSpec((EMB_DIM, HIDDEN), lambda i: (0, 0)),
            pl.BlockSpec((HIDDEN, 1), lambda i: (0, 0)),
            pl.BlockSpec((HIDDEN, STATE), lambda i: (0, 0)),
            pl.BlockSpec((STATE, 1), lambda i: (0, 0)),
        ],
        out_specs=pl.BlockSpec((CBLK, 2 * STATE), lambda i: (i, 0)),
        out_shape=jax.ShapeDtypeStruct((P_ROWS, 2 * STATE), jnp.float32),
    )(xt, xt, W1, b1c, W2, b2c)


REMAP_BLK = 512


def _remap_body(i_ref, o_ref):
    t = i_ref[...]
    pair = t >> (CSH + 1)
    o = t & (2 * CBLK - 1)
    h = o >> CSH
    j = o & (CBLK - 1)
    o_ref[...] = (pair << (CSH + 1)) + 2 * j + h


def _remap(cards_id):
    return pl.pallas_call(
        _remap_body,
        grid=(BATCH // REMAP_BLK,),
        in_specs=[pl.BlockSpec((REMAP_BLK, HIST), lambda i: (i, 0))],
        out_specs=pl.BlockSpec((REMAP_BLK, HIST), lambda i: (i, 0)),
        out_shape=jax.ShapeDtypeStruct((BATCH, HIST), jnp.int32),
    )(cards_id)


def _gather_body(table_hbm, idx_hbm, out_hbm, idx_v, rows_v, sem):
    core = lax.axis_index("c")
    sub = lax.axis_index("s")
    wid = sub * 2 + core
    base = wid * B_PER_W
    pltpu.sync_copy(idx_hbm.at[pl.ds(base, B_PER_W)], idx_v)

    def chunk(c, _):
        off = c * CH
        pltpu.async_copy(
            table_hbm.at[idx_v.at[pl.ds(off, CH)]], rows_v, sem
        ).wait()
        pltpu.sync_copy(rows_v, out_hbm.at[pl.ds(base + off, CH)])
        return ()

    lax.fori_loop(0, N_CH, chunk, (), unroll=False)


def _sc_gather(table, idx):
    mesh = plsc.VectorSubcoreMesh(core_axis_name="c", subcore_axis_name="s")
    return pl.kernel(
        _gather_body,
        out_type=jax.ShapeDtypeStruct((N, STATE), jnp.float32),
        mesh=mesh,
        scratch_types=[
            pltpu.VMEM((B_PER_W,), jnp.int32),
            pltpu.VMEM((CH, STATE), jnp.float32),
            pltpu.SemaphoreType.DMA,
        ],
        compiler_params=pltpu.CompilerParams(use_tc_tiling_on_sc=False),
    )(table, idx)


@jax.jit
def kernel(cards_id, card_embedding, W1, b1, W2, b2):
    xt = jnp.swapaxes(card_embedding, 0, 1)
    packed = _table_mlp(
        xt, W1, b1.reshape(HIDDEN, 1), W2, b2.reshape(STATE, 1)
    )
    tview = packed.reshape(2 * P_ROWS, STATE)
    idx = _remap(cards_id.astype(jnp.int32)).reshape(-1)
    out = _sc_gather(tview, idx)
    out = jax.lax.optimization_barrier(out.reshape(N // 2, 2 * STATE))
    return out.reshape(BATCH, HIST, STATE)


# speedup vs baseline: 1.2954x; 1.0550x over previous
import functools

import jax
import jax.numpy as jnp
from jax import lax
from jax.experimental import pallas as pl
from jax.experimental.pallas import tpu as pltpu
from jax.experimental.pallas import tpu_sc as plsc
from jax.experimental import layout as jax_layout

CARDS_NUM = 1000000
HALF = CARDS_NUM // 2
EMB_DIM = 64
HIDDEN = 128
STATE = 64
BATCH = 4096
HIST = 200

N = BATCH * HIST
NW = 32
B_PER_W = N // NW
CH = 1024
N_CH = B_PER_W // CH

CBLK = 8192
CSH = 13
N_BLK = (CARDS_NUM + 2 * CBLK - 1) // (2 * CBLK)
P_ROWS = N_BLK * CBLK


def _sigmoid(x):
    return 0.5 * jnp.tanh(0.5 * x) + 0.5


def _mlp2_t(x_t, w1, b1c, w2, b2c, eye):
    dn = (((0,), (0,)), ((), ()))
    h_t = _sigmoid(
        lax.dot_general(w1, x_t, dn, preferred_element_type=jnp.float32)
        + b1c
    )
    s_t = _sigmoid(
        lax.dot_general(w2, h_t, dn, preferred_element_type=jnp.float32)
        + b2c
    )
    return lax.dot_general(s_t, eye, dn, preferred_element_type=jnp.float32)


def _table_mlp_body(lo_ref, hi_ref, w1_ref, b1_ref, w2_ref, b2_ref, o_ref):
    w1, w2 = w1_ref[...], w2_ref[...]
    b1c, b2c = b1_ref[...], b2_ref[...]
    eye = jnp.eye(STATE, dtype=jnp.float32)
    o_ref[:, 0:STATE] = _mlp2_t(lo_ref[...], w1, b1c, w2, b2c, eye)
    o_ref[:, STATE:2 * STATE] = _mlp2_t(hi_ref[...], w1, b1c, w2, b2c, eye)


def _table_mlp(xt, W1, b1c, W2, b2c):
    return pl.pallas_call(
        _table_mlp_body,
        grid=(N_BLK,),
        in_specs=[
            pl.BlockSpec((EMB_DIM, CBLK), lambda i: (0, 2 * i)),
            pl.BlockSpec(
                (EMB_DIM, CBLK),
                lambda i: (0, jnp.minimum(2 * i + 1, 2 * N_BLK - 2)),
            ),
            pl.BlockSpec((EMB_DIM, HIDDEN), lambda i: (0, 0)),
            pl.BlockSpec((HIDDEN, 1), lambda i: (0, 0)),
            pl.BlockSpec((HIDDEN, STATE), lambda i: (0, 0)),
            pl.BlockSpec((STATE, 1), lambda i: (0, 0)),
        ],
        out_specs=pl.BlockSpec((CBLK, 2 * STATE), lambda i: (i, 0)),
        out_shape=jax.ShapeDtypeStruct((P_ROWS, 2 * STATE), jnp.float32),
    )(xt, xt, W1, b1c, W2, b2c)


REMAP_BLK = 512


def _remap_body(i_ref, o_ref):
    t = i_ref[...]
    pair = t >> (CSH + 1)
    o = t & (2 * CBLK - 1)
    h = o >> CSH
    j = o & (CBLK - 1)
    o_ref[...] = (pair << (CSH + 1)) + 2 * j + h


def _remap(cards_id):
    return pl.pallas_call(
        _remap_body,
        grid=(BATCH // REMAP_BLK,),
        in_specs=[pl.BlockSpec((REMAP_BLK, HIST), lambda i: (i, 0))],
        out_specs=pl.BlockSpec((REMAP_BLK, HIST), lambda i: (i, 0)),
        out_shape=jax.ShapeDtypeStruct((BATCH, HIST), jnp.int32),
    )(cards_id)


def _gather_body(table_hbm, idx_hbm, out_hbm, idx_v, rows_v, sem):
    core = lax.axis_index("c")
    sub = lax.axis_index("s")
    wid = sub * 2 + core
    base = wid * B_PER_W
    pltpu.sync_copy(idx_hbm.at[pl.ds(base, B_PER_W)], idx_v)

    def chunk(c, _):
        off = c * CH
        pltpu.async_copy(
            table_hbm.at[idx_v.at[pl.ds(off, CH)]], rows_v, sem
        ).wait()
        pltpu.sync_copy(rows_v, out_hbm.at[pl.ds(base + off, CH)])
        return ()

    lax.fori_loop(0, N_CH, chunk, (), unroll=False)


def _sc_gather(table, idx):
    mesh = plsc.VectorSubcoreMesh(core_axis_name="c", subcore_axis_name="s")
    return pl.kernel(
        _gather_body,
        out_type=jax.ShapeDtypeStruct((N, STATE), jnp.float32),
        mesh=mesh,
        scratch_types=[
            pltpu.VMEM((B_PER_W,), jnp.int32),
            pltpu.VMEM((CH, STATE), jnp.float32),
            pltpu.SemaphoreType.DMA,
        ],
        compiler_params=pltpu.CompilerParams(use_tc_tiling_on_sc=False),
    )(table, idx)


@jax.jit
def kernel(cards_id, card_embedding, W1, b1, W2, b2):
    xt = jnp.swapaxes(card_embedding, 0, 1)
    packed = _table_mlp(
        xt, W1, b1.reshape(HIDDEN, 1), W2, b2.reshape(STATE, 1)
    )
    tview = packed.reshape(2 * P_ROWS, STATE)
    idx = _remap(cards_id.astype(jnp.int32)).reshape(-1)
    out = _sc_gather(tview, idx)
    out = jax.lax.optimization_barrier(out.reshape(N // 2, 2 * STATE))
    return out.reshape(BATCH, HIST, STATE)


# speedup vs baseline: 1.3152x; 1.0152x over previous
import jax
import jax.numpy as jnp
from jax import lax
from jax.experimental import pallas as pl
from jax.experimental.pallas import tpu as pltpu
from jax.experimental.pallas import tpu_sc as plsc

CARDS_NUM = 1000000
HALF = CARDS_NUM // 2
EMB_DIM = 64
HIDDEN = 128
STATE = 64
BATCH = 4096
HIST = 200

N = BATCH * HIST
NW = 32
B_PER_W = N // NW
CH = 1024
N_CH = B_PER_W // CH

CBLK = 16384
CSH = 14
N_BLK = (CARDS_NUM + 2 * CBLK - 1) // (2 * CBLK)
P_ROWS = N_BLK * CBLK
MAXB = (CARDS_NUM + CBLK - 1) // CBLK - 1


def _sigmoid(x):
    return 0.5 * jnp.tanh(0.5 * x) + 0.5


def _mlp2_t(x_t, w1, b1c, w2, b2c, eye):
    dn = (((0,), (0,)), ((), ()))
    h_t = _sigmoid(
        lax.dot_general(w1, x_t, dn, preferred_element_type=jnp.float32)
        + b1c
    )
    s_t = _sigmoid(
        lax.dot_general(w2, h_t, dn, preferred_element_type=jnp.float32)
        + b2c
    )
    return lax.dot_general(s_t, eye, dn, preferred_element_type=jnp.float32)


def _table_mlp_body(lo_ref, hi_ref, w1_ref, b1_ref, w2_ref, b2_ref, o_ref):
    w1, w2 = w1_ref[...], w2_ref[...]
    b1c, b2c = b1_ref[...], b2_ref[...]
    eye = jnp.eye(STATE, dtype=jnp.float32)
    o_ref[:, 0:STATE] = _mlp2_t(lo_ref[...], w1, b1c, w2, b2c, eye)
    o_ref[:, STATE:2 * STATE] = _mlp2_t(hi_ref[...], w1, b1c, w2, b2c, eye)


def _table_mlp(xt, W1, b1c, W2, b2c):
    return pl.pallas_call(
        _table_mlp_body,
        grid=(N_BLK,),
        in_specs=[
            pl.BlockSpec((EMB_DIM, CBLK), lambda i: (0, 2 * i)),
            pl.BlockSpec(
                (EMB_DIM, CBLK),
                lambda i: (0, jnp.minimum(2 * i + 1, MAXB)),
            ),
            pl.BlockSpec((EMB_DIM, HIDDEN), lambda i: (0, 0)),
            pl.BlockSpec((HIDDEN, 1), lambda i: (0, 0)),
            pl.BlockSpec((HIDDEN, STATE), lambda i: (0, 0)),
            pl.BlockSpec((STATE, 1), lambda i: (0, 0)),
        ],
        out_specs=pl.BlockSpec((CBLK, 2 * STATE), lambda i: (i, 0)),
        out_shape=jax.ShapeDtypeStruct((P_ROWS, 2 * STATE), jnp.float32),
    )(xt, xt, W1, b1c, W2, b2c)


REMAP_BLK = 512


def _remap_body(i_ref, o_ref):
    t = i_ref[...]
    pair = t >> (CSH + 1)
    o = t & (2 * CBLK - 1)
    h = o >> CSH
    j = o & (CBLK - 1)
    o_ref[...] = (pair << (CSH + 1)) + 2 * j + h


def _remap(cards_id):
    return pl.pallas_call(
        _remap_body,
        grid=(BATCH // REMAP_BLK,),
        in_specs=[pl.BlockSpec((REMAP_BLK, HIST), lambda i: (i, 0))],
        out_specs=pl.BlockSpec((REMAP_BLK, HIST), lambda i: (i, 0)),
        out_shape=jax.ShapeDtypeStruct((BATCH, HIST), jnp.int32),
    )(cards_id)


def _gather_body(table_hbm, idx_hbm, out_hbm, idx_v, rows_v, sem):
    core = lax.axis_index("c")
    sub = lax.axis_index("s")
    wid = sub * 2 + core
    base = wid * B_PER_W
    pltpu.sync_copy(idx_hbm.at[pl.ds(base, B_PER_W)], idx_v)

    def chunk(c, _):
        off = c * CH
        pltpu.async_copy(
            table_hbm.at[idx_v.at[pl.ds(off, CH)]], rows_v, sem
        ).wait()
        pltpu.sync_copy(rows_v, out_hbm.at[pl.ds(base + off, CH)])
        return ()

    lax.fori_loop(0, N_CH, chunk, (), unroll=False)


def _sc_gather(table, idx):
    mesh = plsc.VectorSubcoreMesh(core_axis_name="c", subcore_axis_name="s")
    return pl.kernel(
        _gather_body,
        out_type=jax.ShapeDtypeStruct((N, STATE), jnp.float32),
        mesh=mesh,
        scratch_types=[
            pltpu.VMEM((B_PER_W,), jnp.int32),
            pltpu.VMEM((CH, STATE), jnp.float32),
            pltpu.SemaphoreType.DMA,
        ],
        compiler_params=pltpu.CompilerParams(use_tc_tiling_on_sc=False),
    )(table, idx)


@jax.jit
def kernel(cards_id, card_embedding, W1, b1, W2, b2):
    xt = jnp.swapaxes(card_embedding, 0, 1)
    packed = _table_mlp(
        xt, W1, b1.reshape(HIDDEN, 1), W2, b2.reshape(STATE, 1)
    )
    tview = packed.reshape(2 * P_ROWS, STATE)
    idx = _remap(cards_id.astype(jnp.int32)).reshape(-1)
    out = _sc_gather(tview, idx)
    out = jax.lax.optimization_barrier(out.reshape(N // 2, 2 * STATE))
    return out.reshape(BATCH, HIST, STATE)


# speedup vs baseline: 1.3207x; 1.0042x over previous
import jax
import jax.numpy as jnp
from jax import lax
from jax.experimental import pallas as pl
from jax.experimental.pallas import tpu as pltpu
from jax.experimental.pallas import tpu_sc as plsc

CARDS_NUM = 1000000
HALF = CARDS_NUM // 2
EMB_DIM = 64
HIDDEN = 128
STATE = 64
BATCH = 4096
HIST = 200

N = BATCH * HIST
NW = 32
B_PER_W = N // NW
CH = 1600
N_CH = B_PER_W // CH

CBLK = 16384
CSH = 14
N_BLK = (CARDS_NUM + 2 * CBLK - 1) // (2 * CBLK)
P_ROWS = N_BLK * CBLK
MAXB = (CARDS_NUM + CBLK - 1) // CBLK - 1


def _sigmoid(x):
    return 0.5 * jnp.tanh(0.5 * x) + 0.5


def _mlp2_t(x_t, w1, b1c, w2, b2c, eye):
    dn = (((0,), (0,)), ((), ()))
    h_t = _sigmoid(
        lax.dot_general(w1, x_t, dn, preferred_element_type=jnp.float32)
        + b1c
    )
    s_t = _sigmoid(
        lax.dot_general(w2, h_t, dn, preferred_element_type=jnp.float32)
        + b2c
    )
    return lax.dot_general(s_t, eye, dn, preferred_element_type=jnp.float32)


def _table_mlp_body(lo_ref, hi_ref, w1_ref, b1_ref, w2_ref, b2_ref, o_ref):
    w1, w2 = w1_ref[...], w2_ref[...]
    b1c, b2c = b1_ref[...], b2_ref[...]
    eye = jnp.eye(STATE, dtype=jnp.float32)
    o_ref[:, 0:STATE] = _mlp2_t(lo_ref[...], w1, b1c, w2, b2c, eye)
    o_ref[:, STATE:2 * STATE] = _mlp2_t(hi_ref[...], w1, b1c, w2, b2c, eye)


def _table_mlp(xt, W1, b1c, W2, b2c):
    return pl.pallas_call(
        _table_mlp_body,
        grid=(N_BLK,),
        in_specs=[
            pl.BlockSpec((EMB_DIM, CBLK), lambda i: (0, 2 * i)),
            pl.BlockSpec(
                (EMB_DIM, CBLK),
                lambda i: (0, jnp.minimum(2 * i + 1, MAXB)),
            ),
            pl.BlockSpec((EMB_DIM, HIDDEN), lambda i: (0, 0)),
            pl.BlockSpec((HIDDEN, 1), lambda i: (0, 0)),
            pl.BlockSpec((HIDDEN, STATE), lambda i: (0, 0)),
            pl.BlockSpec((STATE, 1), lambda i: (0, 0)),
        ],
        out_specs=pl.BlockSpec((CBLK, 2 * STATE), lambda i: (i, 0)),
        out_shape=jax.ShapeDtypeStruct((P_ROWS, 2 * STATE), jnp.float32),
    )(xt, xt, W1, b1c, W2, b2c)


REMAP_BLK = 512


def _remap_body(i_ref, o_ref):
    t = i_ref[...]
    pair = t >> (CSH + 1)
    o = t & (2 * CBLK - 1)
    h = o >> CSH
    j = o & (CBLK - 1)
    o_ref[...] = (pair << (CSH + 1)) + 2 * j + h


def _remap(cards_id):
    return pl.pallas_call(
        _remap_body,
        grid=(BATCH // REMAP_BLK,),
        in_specs=[pl.BlockSpec((REMAP_BLK, HIST), lambda i: (i, 0))],
        out_specs=pl.BlockSpec((REMAP_BLK, HIST), lambda i: (i, 0)),
        out_shape=jax.ShapeDtypeStruct((BATCH, HIST), jnp.int32),
    )(cards_id)


def _gather_body(table_hbm, idx_hbm, out_hbm, idx_v, rows_v, sem):
    core = lax.axis_index("c")
    sub = lax.axis_index("s")
    wid = sub * 2 + core
    base = wid * B_PER_W
    pltpu.sync_copy(idx_hbm.at[pl.ds(base, B_PER_W)], idx_v)

    def chunk(c, _):
        off = c * CH
        pltpu.async_copy(
            table_hbm.at[idx_v.at[pl.ds(off, CH)]], rows_v, sem
        ).wait()
        pltpu.sync_copy(rows_v, out_hbm.at[pl.ds(base + off, CH)])
        return ()

    lax.fori_loop(0, N_CH, chunk, (), unroll=False)


def _sc_gather(table, idx):
    mesh = plsc.VectorSubcoreMesh(core_axis_name="c", subcore_axis_name="s")
    return pl.kernel(
        _gather_body,
        out_type=jax.ShapeDtypeStruct((N, STATE), jnp.float32),
        mesh=mesh,
        scratch_types=[
            pltpu.VMEM((B_PER_W,), jnp.int32),
            pltpu.VMEM((CH, STATE), jnp.float32),
            pltpu.SemaphoreType.DMA,
        ],
        compiler_params=pltpu.CompilerParams(use_tc_tiling_on_sc=False),
    )(table, idx)


@jax.jit
def kernel(cards_id, card_embedding, W1, b1, W2, b2):
    xt = jnp.swapaxes(card_embedding, 0, 1)
    packed = _table_mlp(
        xt, W1, b1.reshape(HIDDEN, 1), W2, b2.reshape(STATE, 1)
    )
    tview = packed.reshape(2 * P_ROWS, STATE)
    idx = _remap(cards_id.astype(jnp.int32)).reshape(-1)
    out = _sc_gather(tview, idx)
    out = jax.lax.optimization_barrier(out.reshape(N // 2, 2 * STATE))
    return out.reshape(BATCH, HIST, STATE)
